# Initial kernel scaffold; baseline (speedup 1.0000x reference)
import jax
import jax.numpy as jnp
from jax.experimental import pallas as pl


def kernel(white_features, white_offsets, black_features, black_offsets, stm, ft_weight, ft_bias, W1, b1, W2, b2, Wout, bout):
    raise NotImplementedError("write your pallas kernel here")


# speedup vs baseline: 12.2902x; 12.2902x over previous
import functools

import jax
import jax.numpy as jnp
from jax import lax
from jax.experimental import pallas as pl
from jax.experimental.pallas import tpu as pltpu
from jax.experimental.pallas import tpu_sc as plsc

INPUTS = 40960
L1 = 256
B = 16384

_info = plsc.get_sparse_core_info()
_NC, _NS = _info.num_cores, _info.num_subcores
_NW = _NC * _NS
_BPW = B // _NW
_CH = 128
_NCHUNK = _BPW // _CH


def _sc_gather_body(table_hbm, widx_hbm, bidx_hbm, wout_hbm, bout_hbm,
                    idx_v, rows_v, sem):
    wid = lax.axis_index("s") * _NC + lax.axis_index("c")
    base = wid * _BPW
    for src_hbm, dst_hbm in ((widx_hbm, wout_hbm), (bidx_hbm, bout_hbm)):
        for c in range(_NCHUNK):
            off = base + c * _CH
            pltpu.sync_copy(src_hbm.at[pl.ds(off, _CH)], idx_v)
            pltpu.async_copy(table_hbm.at[idx_v], rows_v, sem).wait()
            pltpu.sync_copy(rows_v, dst_hbm.at[pl.ds(off, _CH)])


def _sc_gather(table, widx, bidx):
    mesh = plsc.VectorSubcoreMesh(core_axis_name="c", subcore_axis_name="s")
    kern = pl.kernel(
        _sc_gather_body,
        mesh=mesh,
        out_type=(
            jax.ShapeDtypeStruct((B, L1), jnp.float32),
            jax.ShapeDtypeStruct((B, L1), jnp.float32),
        ),
        scratch_types=[
            pltpu.VMEM((_CH,), jnp.int32),
            pltpu.VMEM((_CH, L1), jnp.float32),
            pltpu.SemaphoreType.DMA,
        ],
    )
    return kern(table, widx, bidx)


_BLK = 1024


def _mlp_body(wg_ref, bg_ref, stm_ref, fb_ref, w1a_ref, w1b_ref, b1_ref,
              w2_ref, b2_ref, wo_ref, bo_ref, out_ref):
    s = stm_ref[...]
    w = wg_ref[...]
    b = bg_ref[...]
    fb = fb_ref[...]
    xa = jnp.clip(s * w + (1.0 - s) * b + fb, 0.0, 1.0)
    xb = jnp.clip(s * b + (1.0 - s) * w + fb, 0.0, 1.0)
    h1 = jnp.dot(xa, w1a_ref[...], preferred_element_type=jnp.float32)
    h1 += jnp.dot(xb, w1b_ref[...], preferred_element_type=jnp.float32)
    h1 = jnp.clip(h1 + b1_ref[...], 0.0, 1.0)
    h2 = jnp.clip(jnp.dot(h1, w2_ref[...], preferred_element_type=jnp.float32)
                  + b2_ref[...], 0.0, 1.0)
    out_ref[...] = (jnp.dot(h2, wo_ref[...], preferred_element_type=jnp.float32)
                    + bo_ref[...])


def _mlp(wg, bg, stm, ft_bias, W1, b1, W2, b2, Wout, bout):
    w1a = W1[:, :L1].T
    w1b = W1[:, L1:].T
    w2 = W2.T
    wo = Wout.T
    grid = B // _BLK
    rows = lambda i: (i, 0)
    rep = lambda i: (0, 0)
    out = pl.pallas_call(
        _mlp_body,
        grid=(grid,),
        in_specs=[
            pl.BlockSpec((_BLK, L1), rows),
            pl.BlockSpec((_BLK, L1), rows),
            pl.BlockSpec((_BLK, 1), rows),
            pl.BlockSpec((1, L1), rep),
            pl.BlockSpec((L1, 32), rep),
            pl.BlockSpec((L1, 32), rep),
            pl.BlockSpec((1, 32), rep),
            pl.BlockSpec((32, 32), rep),
            pl.BlockSpec((1, 32), rep),
            pl.BlockSpec((32, 1), rep),
            pl.BlockSpec((1, 1), rep),
        ],
        out_specs=pl.BlockSpec((_BLK, 1), rows),
        out_shape=jax.ShapeDtypeStruct((B, 1), jnp.float32),
    )(wg, bg, stm.reshape(B, 1), ft_bias.reshape(1, L1), w1a, w1b,
      b1.reshape(1, 32), w2, b2.reshape(1, 32), wo, bout.reshape(1, 1))
    return out.reshape(B)


def kernel(white_features, white_offsets, black_features, black_offsets, stm,
           ft_weight, ft_bias, W1, b1, W2, b2, Wout, bout):
    widx = white_features.astype(jnp.int32)
    bidx = black_features.astype(jnp.int32)
    wg, bg = _sc_gather(ft_weight, widx, bidx)
    return _mlp(wg, bg, stm, ft_bias, W1, b1, W2, b2, Wout, bout)


# speedup vs baseline: 14.2734x; 1.1614x over previous
import functools

import jax
import jax.numpy as jnp
from jax import lax
from jax.experimental import pallas as pl
from jax.experimental.pallas import tpu as pltpu
from jax.experimental.pallas import tpu_sc as plsc

INPUTS = 40960
L1 = 256
B = 16384

_info = plsc.get_sparse_core_info()
_NC, _NS = _info.num_cores, _info.num_subcores
_NW = _NC * _NS
_BPW = B // _NW
_CH = 128
_NCHUNK = _BPW // _CH


_NBUF = 3


def _sc_gather_body(table_hbm, widx_hbm, bidx_hbm, wout_hbm, bout_hbm,
                    widx_v, bidx_v, rows0, rows1, rows2, gsem, wsem):
    wid = lax.axis_index("s") * _NC + lax.axis_index("c")
    base = wid * _BPW
    rowsb = (rows0, rows1, rows2)
    pltpu.sync_copy(widx_hbm.at[pl.ds(base, _BPW)], widx_v)
    pltpu.sync_copy(bidx_hbm.at[pl.ds(base, _BPW)], bidx_v)
    work = []
    for idx_v, dst_hbm in ((widx_v, wout_hbm), (bidx_v, bout_hbm)):
        for c in range(_NCHUNK):
            work.append((idx_v, dst_hbm, c * _CH))
    writebacks = []
    for i, (idx_v, dst_hbm, off) in enumerate(work):
        buf = rowsb[i % _NBUF]
        if i >= _NBUF:
            writebacks[i - _NBUF].wait()
        pltpu.async_copy(table_hbm.at[idx_v.at[pl.ds(off, _CH)]], buf,
                         gsem).wait()
        writebacks.append(
            pltpu.async_copy(buf, dst_hbm.at[pl.ds(base + off, _CH)], wsem))
    for wb in writebacks[-_NBUF:]:
        wb.wait()


def _sc_gather(table, widx, bidx):
    mesh = plsc.VectorSubcoreMesh(core_axis_name="c", subcore_axis_name="s")
    kern = pl.kernel(
        _sc_gather_body,
        mesh=mesh,
        out_type=(
            jax.ShapeDtypeStruct((B, L1), jnp.float32),
            jax.ShapeDtypeStruct((B, L1), jnp.float32),
        ),
        scratch_types=[
            pltpu.VMEM((_BPW,), jnp.int32),
            pltpu.VMEM((_BPW,), jnp.int32),
            pltpu.VMEM((_CH, L1), jnp.float32),
            pltpu.VMEM((_CH, L1), jnp.float32),
            pltpu.VMEM((_CH, L1), jnp.float32),
            pltpu.SemaphoreType.DMA,
            pltpu.SemaphoreType.DMA,
        ],
    )
    return kern(table, widx, bidx)


_BLK = 2048


def _mlp_body(wg_ref, bg_ref, stm_ref, fb_ref, w1a_ref, w1b_ref, b1_ref,
              w2_ref, b2_ref, wo_ref, bo_ref, out_ref):
    s = stm_ref[...]
    w = wg_ref[...]
    b = bg_ref[...]
    fb = fb_ref[...]
    xa = jnp.clip(s * w + (1.0 - s) * b + fb, 0.0, 1.0)
    xb = jnp.clip(s * b + (1.0 - s) * w + fb, 0.0, 1.0)
    h1 = jnp.dot(xa, w1a_ref[...], preferred_element_type=jnp.float32)
    h1 += jnp.dot(xb, w1b_ref[...], preferred_element_type=jnp.float32)
    h1 = jnp.clip(h1 + b1_ref[...], 0.0, 1.0)
    h2 = jnp.clip(jnp.dot(h1, w2_ref[...], preferred_element_type=jnp.float32)
                  + b2_ref[...], 0.0, 1.0)
    out_ref[...] = (jnp.dot(h2, wo_ref[...], preferred_element_type=jnp.float32)
                    + bo_ref[...])


def _mlp(wg, bg, stm, ft_bias, W1, b1, W2, b2, Wout, bout):
    w1a = W1[:, :L1].T
    w1b = W1[:, L1:].T
    w2 = W2.T
    wo = Wout.T
    grid = B // _BLK
    rows = lambda i: (i, 0)
    rep = lambda i: (0, 0)
    out = pl.pallas_call(
        _mlp_body,
        grid=(grid,),
        in_specs=[
            pl.BlockSpec((_BLK, L1), rows),
            pl.BlockSpec((_BLK, L1), rows),
            pl.BlockSpec((_BLK, 1), rows),
            pl.BlockSpec((1, L1), rep),
            pl.BlockSpec((L1, 32), rep),
            pl.BlockSpec((L1, 32), rep),
            pl.BlockSpec((1, 32), rep),
            pl.BlockSpec((32, 32), rep),
            pl.BlockSpec((1, 32), rep),
            pl.BlockSpec((32, 1), rep),
            pl.BlockSpec((1, 1), rep),
        ],
        out_specs=pl.BlockSpec((_BLK, 1), rows),
        out_shape=jax.ShapeDtypeStruct((B, 1), jnp.float32),
    )(wg, bg, stm.reshape(B, 1), ft_bias.reshape(1, L1), w1a, w1b,
      b1.reshape(1, 32), w2, b2.reshape(1, 32), wo, bout.reshape(1, 1))
    return out.reshape(B)


def kernel(white_features, white_offsets, black_features, black_offsets, stm,
           ft_weight, ft_bias, W1, b1, W2, b2, Wout, bout):
    widx = white_features.astype(jnp.int32)
    bidx = black_features.astype(jnp.int32)
    wg, bg = _sc_gather(ft_weight, widx, bidx)
    return _mlp(wg, bg, stm, ft_bias, W1, b1, W2, b2, Wout, bout)


# speedup vs baseline: 15.5204x; 1.0874x over previous
import functools

import jax
import jax.numpy as jnp
from jax import lax
from jax.experimental import pallas as pl
from jax.experimental.pallas import tpu as pltpu
from jax.experimental.pallas import tpu_sc as plsc

INPUTS = 40960
L1 = 256
B = 16384

_info = plsc.get_sparse_core_info()
_NC, _NS = _info.num_cores, _info.num_subcores
_NW = _NC * _NS
_BPW = B // _NW
_CH = 128
_NCHUNK = _BPW // _CH


_NBUF = 3


def _sc_gather_body(table_hbm, widx_hbm, bidx_hbm, wout_hbm, bout_hbm,
                    widx_v, bidx_v, rows0, rows1, rows2, gsem, wsem):
    wid = lax.axis_index("s") * _NC + lax.axis_index("c")
    base = wid * _BPW
    rowsb = (rows0, rows1, rows2)
    pltpu.sync_copy(widx_hbm.at[pl.ds(base, _BPW)], widx_v)
    pltpu.sync_copy(bidx_hbm.at[pl.ds(base, _BPW)], bidx_v)
    work = []
    for idx_v, dst_hbm in ((widx_v, wout_hbm), (bidx_v, bout_hbm)):
        for c in range(_NCHUNK):
            work.append((idx_v, dst_hbm, c * _CH))
    writebacks = []
    for i, (idx_v, dst_hbm, off) in enumerate(work):
        buf = rowsb[i % _NBUF]
        if i >= _NBUF:
            writebacks[i - _NBUF].wait()
        pltpu.async_copy(table_hbm.at[idx_v.at[pl.ds(off, _CH)]], buf,
                         gsem).wait()
        writebacks.append(
            pltpu.async_copy(buf, dst_hbm.at[pl.ds(base + off, _CH)], wsem))
    for wb in writebacks[-_NBUF:]:
        wb.wait()


def _sc_gather(table, widx, bidx):
    mesh = plsc.VectorSubcoreMesh(core_axis_name="c", subcore_axis_name="s")
    kern = pl.kernel(
        _sc_gather_body,
        mesh=mesh,
        out_type=(
            jax.ShapeDtypeStruct((B, L1), jnp.float32),
            jax.ShapeDtypeStruct((B, L1), jnp.float32),
        ),
        scratch_types=[
            pltpu.VMEM((_BPW,), jnp.int32),
            pltpu.VMEM((_BPW,), jnp.int32),
            pltpu.VMEM((_CH, L1), jnp.float32),
            pltpu.VMEM((_CH, L1), jnp.float32),
            pltpu.VMEM((_CH, L1), jnp.float32),
            pltpu.SemaphoreType.DMA,
            pltpu.SemaphoreType.DMA,
        ],
    )
    return kern(table, widx, bidx)


_BLK = 2048


_CONTRACT_MINOR = (((1,), (1,)), ((), ()))


def _mlp_body(wg_ref, bg_ref, stm_ref, fb_ref, w1a_ref, w1b_ref, b1_ref,
              w2_ref, b2_ref, wo_ref, bo_ref, out_ref):
    s = jnp.reshape(stm_ref[...], (_BLK, 1))
    w = wg_ref[...]
    b = bg_ref[...]
    fb = fb_ref[...]
    xa = jnp.clip(s * w + (1.0 - s) * b + fb, 0.0, 1.0)
    xb = jnp.clip(s * b + (1.0 - s) * w + fb, 0.0, 1.0)
    h1 = lax.dot_general(xa, w1a_ref[...], _CONTRACT_MINOR,
                         preferred_element_type=jnp.float32)
    h1 += lax.dot_general(xb, w1b_ref[...], _CONTRACT_MINOR,
                          preferred_element_type=jnp.float32)
    h1 = jnp.clip(h1 + b1_ref[...], 0.0, 1.0)
    h2 = jnp.clip(lax.dot_general(h1, w2_ref[...], _CONTRACT_MINOR,
                                  preferred_element_type=jnp.float32)
                  + b2_ref[...], 0.0, 1.0)
    o = jnp.dot(h2, wo_ref[...], preferred_element_type=jnp.float32) \
        + bo_ref[...]
    out_ref[...] = jnp.reshape(o, (1, _BLK))


def _mlp(wg, bg, stm, ft_bias, W1, b1, W2, b2, Wout, bout):
    grid = B // _BLK
    rows = lambda i: (i, 0)
    cols = lambda i: (0, i)
    rep = lambda i: (0, 0)
    out = pl.pallas_call(
        _mlp_body,
        grid=(grid,),
        in_specs=[
            pl.BlockSpec((_BLK, L1), rows),
            pl.BlockSpec((_BLK, L1), rows),
            pl.BlockSpec((1, _BLK), cols),
            pl.BlockSpec((1, L1), rep),
            pl.BlockSpec((32, L1), rep),
            pl.BlockSpec((32, L1), rep),
            pl.BlockSpec((1, 32), rep),
            pl.BlockSpec((32, 32), rep),
            pl.BlockSpec((1, 32), rep),
            pl.BlockSpec((32, 1), rep),
            pl.BlockSpec((1, 1), rep),
        ],
        out_specs=pl.BlockSpec((1, _BLK), cols),
        out_shape=jax.ShapeDtypeStruct((1, B), jnp.float32),
    )(wg, bg, stm.reshape(1, B), ft_bias.reshape(1, L1), W1[:, :L1],
      W1[:, L1:], b1.reshape(1, 32), W2, b2.reshape(1, 32), Wout.T,
      bout.reshape(1, 1))
    return out.reshape(B)


def kernel(white_features, white_offsets, black_features, black_offsets, stm,
           ft_weight, ft_bias, W1, b1, W2, b2, Wout, bout):
    widx = white_features.astype(jnp.int32)
    bidx = black_features.astype(jnp.int32)
    wg, bg = _sc_gather(ft_weight, widx, bidx)
    return _mlp(wg, bg, stm, ft_bias, W1, b1, W2, b2, Wout, bout)


# speedup vs baseline: 15.6436x; 1.0079x over previous
import functools

import jax
import jax.numpy as jnp
from jax import lax
from jax.experimental import pallas as pl
from jax.experimental.pallas import tpu as pltpu
from jax.experimental.pallas import tpu_sc as plsc

INPUTS = 40960
L1 = 256
B = 16384

_info = plsc.get_sparse_core_info()
_NC, _NS = _info.num_cores, _info.num_subcores
_NW = _NC * _NS
_BPW = B // _NW
_CH = 128
_NCHUNK = _BPW // _CH


_NBUF = 3


def _sc_gather_body(table_hbm, widx_hbm, bidx_hbm, wout_hbm, bout_hbm,
                    widx_v, bidx_v, rows0, rows1, rows2, gsem, wsem, isem):
    wid = lax.axis_index("s") * _NC + lax.axis_index("c")
    base = wid * _BPW
    rowsb = (rows0, rows1, rows2)
    widx_cp = pltpu.async_copy(widx_hbm.at[pl.ds(base, _BPW)], widx_v, isem)
    bidx_cp = pltpu.async_copy(bidx_hbm.at[pl.ds(base, _BPW)], bidx_v, isem)
    widx_cp.wait()
    bidx_cp.wait()
    work = []
    for idx_v, dst_hbm in ((widx_v, wout_hbm), (bidx_v, bout_hbm)):
        for c in range(_NCHUNK):
            work.append((idx_v, dst_hbm, c * _CH))
    writebacks = []
    for i, (idx_v, dst_hbm, off) in enumerate(work):
        buf = rowsb[i % _NBUF]
        if i >= _NBUF:
            writebacks[i - _NBUF].wait()
        pltpu.async_copy(table_hbm.at[idx_v.at[pl.ds(off, _CH)]], buf,
                         gsem).wait()
        writebacks.append(
            pltpu.async_copy(buf, dst_hbm.at[pl.ds(base + off, _CH)], wsem))
    for wb in writebacks[-_NBUF:]:
        wb.wait()


def _sc_gather(table, widx, bidx):
    mesh = plsc.VectorSubcoreMesh(core_axis_name="c", subcore_axis_name="s")
    kern = pl.kernel(
        _sc_gather_body,
        mesh=mesh,
        out_type=(
            jax.ShapeDtypeStruct((B, L1), jnp.float32),
            jax.ShapeDtypeStruct((B, L1), jnp.float32),
        ),
        scratch_types=[
            pltpu.VMEM((_BPW,), jnp.int32),
            pltpu.VMEM((_BPW,), jnp.int32),
            pltpu.VMEM((_CH, L1), jnp.float32),
            pltpu.VMEM((_CH, L1), jnp.float32),
            pltpu.VMEM((_CH, L1), jnp.float32),
            pltpu.SemaphoreType.DMA,
            pltpu.SemaphoreType.DMA,
            pltpu.SemaphoreType.DMA,
        ],
    )
    return kern(table, widx, bidx)


_BLK = 2048


_CONTRACT_MINOR = (((1,), (1,)), ((), ()))


def _mlp_body(wg_ref, bg_ref, stm_ref, fb_ref, w1a_ref, w1b_ref, b1_ref,
              w2_ref, b2_ref, wo_ref, bo_ref, out_ref):
    s = jnp.reshape(stm_ref[...], (_BLK, 1))
    w = wg_ref[...]
    b = bg_ref[...]
    fb = fb_ref[...]
    t = s * (w - b)
    xa = jnp.clip(b + fb + t, 0.0, 1.0).astype(jnp.bfloat16)
    xb = jnp.clip(w + fb - t, 0.0, 1.0).astype(jnp.bfloat16)
    h1 = lax.dot_general(xa, w1a_ref[...], _CONTRACT_MINOR,
                         preferred_element_type=jnp.float32)
    h1 += lax.dot_general(xb, w1b_ref[...], _CONTRACT_MINOR,
                          preferred_element_type=jnp.float32)
    h1 = jnp.clip(h1 + b1_ref[...], 0.0, 1.0)
    h2 = jnp.clip(lax.dot_general(h1, w2_ref[...], _CONTRACT_MINOR,
                                  preferred_element_type=jnp.float32)
                  + b2_ref[...], 0.0, 1.0)
    o = jnp.dot(h2, wo_ref[...], preferred_element_type=jnp.float32) \
        + bo_ref[...]
    out_ref[...] = jnp.reshape(o, (1, _BLK))


def _mlp(wg, bg, stm, ft_bias, W1, b1, W2, b2, Wout, bout):
    grid = B // _BLK
    rows = lambda i: (i, 0)
    cols = lambda i: (0, i)
    rep = lambda i: (0, 0)
    out = pl.pallas_call(
        _mlp_body,
        grid=(grid,),
        in_specs=[
            pl.BlockSpec((_BLK, L1), rows),
            pl.BlockSpec((_BLK, L1), rows),
            pl.BlockSpec((1, _BLK), cols),
            pl.BlockSpec((1, L1), rep),
            pl.BlockSpec((32, L1), rep),
            pl.BlockSpec((32, L1), rep),
            pl.BlockSpec((1, 32), rep),
            pl.BlockSpec((32, 32), rep),
            pl.BlockSpec((1, 32), rep),
            pl.BlockSpec((32, 1), rep),
            pl.BlockSpec((1, 1), rep),
        ],
        out_specs=pl.BlockSpec((1, _BLK), cols),
        out_shape=jax.ShapeDtypeStruct((1, B), jnp.float32),
    )(wg, bg, stm.reshape(1, B), ft_bias.reshape(1, L1),
      W1[:, :L1].astype(jnp.bfloat16), W1[:, L1:].astype(jnp.bfloat16),
      b1.reshape(1, 32), W2, b2.reshape(1, 32), Wout.T,
      bout.reshape(1, 1))
    return out.reshape(B)


def kernel(white_features, white_offsets, black_features, black_offsets, stm,
           ft_weight, ft_bias, W1, b1, W2, b2, Wout, bout):
    widx = white_features.astype(jnp.int32)
    bidx = black_features.astype(jnp.int32)
    wg, bg = _sc_gather(ft_weight, widx, bidx)
    return _mlp(wg, bg, stm, ft_bias, W1, b1, W2, b2, Wout, bout)
